# baseline (speedup 1.0000x reference)
import functools

import jax
import jax.numpy as jnp
from jax.experimental import pallas as pl
from jax.experimental.pallas import tpu as pltpu

N = 10000
ROW_BLK = 400
GRID = N // ROW_BLK


def _gconv_body(h_ref, w_ref, b_ref, a_ref, o_ref, *, relu, windowmax):
    t = jnp.dot(a_ref[...], h_ref[...])
    out = jnp.dot(t, w_ref[...]) + b_ref[...]
    if relu:
        out = jnp.maximum(out, 0.0)
    if windowmax:
        z1 = jnp.zeros((out.shape[0], 1), jnp.float32)
        z2 = jnp.zeros((out.shape[0], 2), jnp.float32)
        hp = jnp.concatenate([z1, out, z2], axis=1)
        s = hp[:, 0:256]
        for w in range(1, 5):
            s = jnp.maximum(s, hp[:, w:w + 256])
        out = s.reshape(out.shape[0], 128, 2)[:, :, 0]
    o_ref[...] = out


def _gconv(h, w, b, edges, kout, relu, windowmax, row_blk=ROW_BLK):
    kin = h.shape[1]
    body = functools.partial(_gconv_body, relu=relu, windowmax=windowmax)
    return pl.pallas_call(
        body,
        grid=(N // row_blk,),
        in_specs=[
            pl.BlockSpec((N, kin), lambda i: (0, 0)),
            pl.BlockSpec((kin, w.shape[1]), lambda i: (0, 0)),
            pl.BlockSpec((1, w.shape[1]), lambda i: (0, 0)),
            pl.BlockSpec((row_blk, N), lambda i: (i, 0)),
        ],
        out_specs=pl.BlockSpec((row_blk, kout), lambda i: (i, 0)),
        out_shape=jax.ShapeDtypeStruct((N, kout), jnp.float32),
    )(h, w, b, edges)


def _conv3(x, w0, w1, w2, b):
    cin = x.shape[0]
    z = jnp.zeros((cin, 1), jnp.float32)
    xp = jnp.concatenate([z, x, z], axis=1)
    out = b + jnp.dot(w0, xp[:, 0:64])
    out = out + jnp.dot(w1, xp[:, 1:65])
    out = out + jnp.dot(w2, xp[:, 2:66])
    return out


def _final_body(x3_ref, labels_ref, emb_ref, wfc_ref, fcb_ref, rb_refs,
                loss_ref, preds_ref):
    y = emb_ref[...]
    for (w10, w11, w12, b1r, w20, w21, w22, b2r, ws, bs) in rb_refs:
        h = _conv3(y, w10[...], w11[...], w12[...], b1r[...])
        h = jnp.maximum(h, 0.0)
        h = _conv3(h, w20[...], w21[...], w22[...], b2r[...])
        s = bs[...] + jnp.dot(ws[...], y)
        y = jnp.maximum(h + s, 0.0)
    yv = jnp.zeros((1, 64), jnp.float32)
    for c in range(9):
        yv = yv + jnp.dot(y[c:c + 1, :], wfc_ref[64 * c:64 * (c + 1), :],
                          precision=jax.lax.Precision.HIGHEST)
    yv = yv + fcb_ref[...]

    x3 = x3_ref[...]
    num = jnp.sum(x3 * yv, axis=1, keepdims=True)
    xn = jnp.sqrt(jnp.sum(x3 * x3, axis=1, keepdims=True))
    yn = jnp.sqrt(jnp.sum(yv * yv))
    sim = num / jnp.maximum(xn * yn, 1e-8)

    big = jnp.int32(2 ** 30)
    ri = jax.lax.broadcasted_iota(jnp.int32, (N, 1), 0)
    m = jnp.max(sim)
    pred = jnp.min(jnp.where(sim >= m, ri, big))
    lab_v = labels_ref[...]
    li = jax.lax.broadcasted_iota(jnp.int32, (1, N), 1)
    lm = jnp.max(lab_v)
    lab = jnp.min(jnp.where(lab_v >= lm, li, big))
    lse = m + jnp.log(jnp.sum(jnp.exp(sim - m)))
    tgt = jnp.sum(jnp.where(ri == lab, sim, 0.0))
    loss_ref[...] = jnp.reshape(lse - tgt, (1, 1))
    preds_ref[...] = jnp.reshape(pred, (1, 1))


def _final(x3, labels, emb, wfc, fcb, rbs):
    n_rb = len(rbs)

    def body(*refs):
        x3_ref, labels_ref, emb_ref, wfc_ref, fcb_ref = refs[:5]
        flat = refs[5:5 + 10 * n_rb]
        rb_refs = [tuple(flat[10 * i:10 * (i + 1)]) for i in range(n_rb)]
        loss_ref, preds_ref = refs[5 + 10 * n_rb:]
        _final_body(x3_ref, labels_ref, emb_ref, wfc_ref, fcb_ref, rb_refs,
                    loss_ref, preds_ref)

    flat_rb = [t for rb in rbs for t in rb]
    return pl.pallas_call(
        body,
        out_shape=(jax.ShapeDtypeStruct((1, 1), jnp.float32),
                   jax.ShapeDtypeStruct((1, 1), jnp.int32)),
    )(x3, labels, emb, wfc, fcb, *flat_rb)


def kernel(edges, embeddings, labels, node_embed, W1, b1, W2, b2, W3, b3,
           rb1_w1, rb1_b1, rb1_w2, rb1_b2, rb1_ws, rb1_bs,
           rb2_w1, rb2_b1, rb2_w2, rb2_b2, rb2_ws, rb2_bs,
           rb3_w1, rb3_b1, rb3_w2, rb3_b2, rb3_ws, rb3_bs,
           rb4_w1, rb4_b1, rb4_w2, rb4_b2, rb4_ws, rb4_bs,
           fc1_w, fc1_b):
    b1r = b1.reshape(1, -1)
    b2r = b2.reshape(1, -1)
    b3r = b3.reshape(1, -1)
    f = jnp.arange(576)
    wfc = jnp.zeros((576, 64), jnp.float32).at[f, f // 9].set(fc1_w[f % 9, 0])
    fcb = fc1_b.reshape(1, 1)

    def split_rb(w1_, b1_, w2_, b2_, ws_, bs_):
        return (w1_[:, :, 0], w1_[:, :, 1], w1_[:, :, 2], b1_.reshape(-1, 1),
                w2_[:, :, 0], w2_[:, :, 1], w2_[:, :, 2], b2_.reshape(-1, 1),
                ws_[:, :, 0], bs_.reshape(-1, 1))

    rbs = [split_rb(rb1_w1, rb1_b1, rb1_w2, rb1_b2, rb1_ws, rb1_bs),
           split_rb(rb2_w1, rb2_b1, rb2_w2, rb2_b2, rb2_ws, rb2_bs),
           split_rb(rb3_w1, rb3_b1, rb3_w2, rb3_b2, rb3_ws, rb3_bs),
           split_rb(rb4_w1, rb4_b1, rb4_w2, rb4_b2, rb4_ws, rb4_bs)]

    h1 = _gconv(node_embed, W1, b1r, edges, 256, relu=True, windowmax=False)
    p2 = _gconv(h1, W2, b2r, edges, 128, relu=True, windowmax=True,
                row_blk=200)
    x3 = _gconv(p2, W3, b3r, edges, 64, relu=False, windowmax=False)

    loss, preds = _final(x3, labels, embeddings.astype(jnp.float32),
                         wfc, fcb, rbs)
    return loss[0, 0], preds
